# baseline (speedup 1.0000x reference)
import functools

import jax
import jax.numpy as jnp
from jax import lax
from jax.experimental import pallas as pl
from jax.experimental.pallas import tpu as pltpu
from jax.experimental.pallas import tpu_sc as plsc

N = 10000
E = 320000
D = 128
H = 64
G = 128

NC = 2
NS = 16
NW = NC * NS

NPAD = 10240
ROWS_PER_TILE = NPAD // NS

CHUNK = 128
NCHUNK = 80
EPAD = NW * NCHUNK * CHUNK

_SC_MESH = plsc.VectorSubcoreMesh(
    core_axis_name="c", subcore_axis_name="s", num_cores=NC, num_subcores=NS)


@functools.partial(
    pl.kernel,
    out_type=jax.ShapeDtypeStruct((NC, NPAD), jnp.float32),
    mesh=_SC_MESH,
    scratch_types=[
        pltpu.VMEM((NCHUNK, CHUNK), jnp.int32),
        pltpu.VMEM((CHUNK,), jnp.float32),
        pltpu.VMEM_SHARED((NPAD,), jnp.float32),
        pltpu.SemaphoreType.DMA,
    ],
    compiler_params=pltpu.CompilerParams(use_tc_tiling_on_sc=False),
)
def _sc_deg(dst_hbm, zeros_hbm, ones_hbm, out_hbm, dst_v, ones_v, acc, sem):
    cid = lax.axis_index("c")
    sid = lax.axis_index("s")
    wid = cid * NS + sid
    row0 = pl.multiple_of(sid * ROWS_PER_TILE, 8)
    pltpu.sync_copy(zeros_hbm, acc.at[pl.ds(row0, ROWS_PER_TILE)])
    pltpu.sync_copy(ones_hbm, ones_v)
    pltpu.sync_copy(dst_hbm.at[wid], dst_v)
    plsc.subcore_barrier()

    def wait_one():
        pltpu.make_async_copy(ones_v, acc.at[dst_v.at[0]], sem).wait()

    def body(j, carry):
        pltpu.async_copy(ones_v, acc.at[dst_v.at[j]], sem, add=True)
        return carry

    lax.fori_loop(0, 8, body, 0)

    def body2(j, carry):
        pltpu.async_copy(ones_v, acc.at[dst_v.at[j]], sem, add=True)
        wait_one()
        return carry

    lax.fori_loop(8, NCHUNK, body2, 0)

    def drain(j, carry):
        wait_one()
        return carry

    lax.fori_loop(0, 8, drain, 0)
    plsc.subcore_barrier()
    pltpu.sync_copy(acc.at[pl.ds(row0, ROWS_PER_TILE)],
                    out_hbm.at[cid, pl.ds(row0, ROWS_PER_TILE)])


@functools.partial(
    pl.kernel,
    out_type=jax.ShapeDtypeStruct((NC, NPAD, H), jnp.float32),
    mesh=_SC_MESH,
    scratch_types=[
        pltpu.VMEM((NCHUNK, CHUNK), jnp.int32),
        pltpu.VMEM((NCHUNK, CHUNK), jnp.int32),
        [pltpu.VMEM((CHUNK, H), jnp.float32)] * 8,
        [pltpu.SemaphoreType.DMA] * 8,
        [pltpu.SemaphoreType.DMA] * 8,
        pltpu.VMEM_SHARED((NPAD, H), jnp.float32),
    ],
    compiler_params=pltpu.CompilerParams(use_tc_tiling_on_sc=False),
)
def _sc_agg(t_hbm, src_hbm, dst_hbm, zeros_hbm, out_hbm,
            src_v, dst_v, bufs, gs, ss, acc):
    cid = lax.axis_index("c")
    sid = lax.axis_index("s")
    wid = cid * NS + sid
    row0 = pl.multiple_of(sid * ROWS_PER_TILE, 8)
    pltpu.sync_copy(zeros_hbm, acc.at[pl.ds(row0, ROWS_PER_TILE)])
    pltpu.sync_copy(src_hbm.at[wid], src_v)
    pltpu.sync_copy(dst_hbm.at[wid], dst_v)
    plsc.subcore_barrier()

    def gather(j, k):
        pltpu.async_copy(t_hbm.at[src_v.at[j]], bufs[k], gs[k])

    def scatter(j, k):
        pltpu.async_copy(bufs[k], acc.at[dst_v.at[j]], ss[k], add=True)

    def wait(k, sem):
        pltpu.make_async_copy(t_hbm.at[src_v.at[0]], bufs[k], sem).wait()

    for k in range(4):
        gather(k, k)
    for p in range(8):
        k = p % 8
        k4 = (p + 4) % 8
        wait(k, gs[k])
        scatter(p, k)
        if p >= 4:
            wait(k4, ss[k4])
        gather(p + 4, k4)

    def round_body(i, carry):
        p0 = 8 * i
        for k in range(8):
            k4 = (k + 4) % 8
            wait(k, gs[k])
            scatter(p0 + k, k)
            wait(k4, ss[k4])
            gather(p0 + k + 4, k4)
        return carry

    lax.fori_loop(1, NCHUNK // 8 - 1, round_body, 0)

    for p in range(NCHUNK - 8, NCHUNK):
        k = p % 8
        k4 = (p + 4) % 8
        wait(k, gs[k])
        scatter(p, k)
        wait(k4, ss[k4])
        if p + 4 < NCHUNK:
            gather(p + 4, k4)
    for c in range(NCHUNK - 4, NCHUNK):
        wait(c % 8, ss[c % 8])
    plsc.subcore_barrier()
    pltpu.sync_copy(acc.at[pl.ds(row0, ROWS_PER_TILE)],
                    out_hbm.at[cid, pl.ds(row0, ROWS_PER_TILE)])


def _tc1_body(deg0, deg1, x, W1, dis_o, t1p_o):
    deg = deg0[...] + deg1[...] + 1.0
    dis = lax.rsqrt(deg)
    dis_o[...] = dis
    t = lax.dot_general(x[...].astype(jnp.bfloat16), W1[...].astype(jnp.bfloat16),
                        (((1,), (1,)), ((), ())),
                        preferred_element_type=jnp.float32)
    t1p_o[...] = t * dis


def _tc2_body(agg0, agg1, t1p, dis, b1, gamma1, beta1, W2, t2p_o):
    d = dis[...]
    z = (agg0[...] + agg1[...] + t1p[...]) * d + b1[...]
    mu = jnp.mean(z, axis=0, keepdims=True)
    var = jnp.mean((z - mu) ** 2, axis=0, keepdims=True)
    h = jnp.maximum((z - mu) / jnp.sqrt(var + 1e-5) * gamma1[...] + beta1[...], 0.0)
    t2 = lax.dot_general(h.astype(jnp.bfloat16), W2[...].astype(jnp.bfloat16),
                         (((1,), (1,)), ((), ())),
                         preferred_element_type=jnp.float32)
    t2p_o[...] = t2 * d


def _tc3_body(agg0, agg1, t2p, dis, b2, gamma2, beta2, batch, Wl, bl, out_o):
    z = (agg0[...] + agg1[...] + t2p[...]) * dis[...] + b2[...]
    mu = jnp.mean(z, axis=0, keepdims=True)
    var = jnp.mean((z - mu) ** 2, axis=0, keepdims=True)
    h = jnp.maximum((z - mu) / jnp.sqrt(var + 1e-5) * gamma2[...] + beta2[...], 0.0)
    gid = lax.broadcasted_iota(jnp.int32, (G, N), 0)
    onehot = (batch[...] == gid).astype(jnp.float32)
    sums = lax.dot_general(onehot, h, (((1,), (0,)), ((), ())),
                           preferred_element_type=jnp.float32,
                        precision=lax.Precision.HIGHEST)
    cnt = jnp.sum(onehot, axis=1, keepdims=True)
    mean = sums / jnp.maximum(cnt, 1.0)
    mb = mean.astype(jnp.bfloat16).astype(jnp.float32)
    wb = Wl[...].astype(jnp.bfloat16).astype(jnp.float32)
    out_o[...] = jnp.sum(mb * wb, axis=1, keepdims=True) + bl[...]


def kernel(x, edge_index, batch, W1, b1, gamma1, beta1, W2, b2, gamma2, beta2,
           Wl, bl):
    pad = EPAD - E
    src = jnp.concatenate(
        [edge_index[0], jnp.zeros((pad,), jnp.int32)]).reshape(NW, NCHUNK, CHUNK)
    dst = jnp.concatenate(
        [edge_index[1], jnp.full((pad,), N, jnp.int32)]).reshape(NW, NCHUNK, CHUNK)
    zeros_row = jnp.zeros((ROWS_PER_TILE,), jnp.float32)
    zeros_mat = jnp.zeros((ROWS_PER_TILE, H), jnp.float32)
    ones_chunk = jnp.ones((CHUNK,), jnp.float32)

    degp = _sc_deg(dst, zeros_row, ones_chunk)
    deg0 = degp[0, :N].reshape(N, 1)
    deg1 = degp[1, :N].reshape(N, 1)

    dis, t1p = pl.pallas_call(
        _tc1_body,
        out_shape=[
            jax.ShapeDtypeStruct((N, 1), jnp.float32),
            jax.ShapeDtypeStruct((N, H), jnp.float32),
        ],
    )(deg0, deg1, x, W1)

    agg1 = _sc_agg(t1p, src, dst, zeros_mat)

    t2p = pl.pallas_call(
        _tc2_body,
        out_shape=jax.ShapeDtypeStruct((N, H), jnp.float32),
    )(agg1[0, :N], agg1[1, :N], t1p, dis, b1, gamma1, beta1, W2)

    agg2 = _sc_agg(t2p, src, dst, zeros_mat)

    out = pl.pallas_call(
        _tc3_body,
        out_shape=jax.ShapeDtypeStruct((G, 1), jnp.float32),
    )(agg2[0, :N], agg2[1, :N], t2p, dis, b2, gamma2, beta2,
      batch.reshape(1, N), Wl, bl)
    return out


# speedup vs baseline: 2.3419x; 2.3419x over previous
import functools

import jax
import jax.numpy as jnp
from jax import lax
from jax.experimental import pallas as pl
from jax.experimental.pallas import tpu as pltpu
from jax.experimental.pallas import tpu_sc as plsc

N = 10000
E = 320000
D = 128
H = 64
G = 128

NC = 2
NS = 16
NW = NC * NS

NPAD = 10240
ROWS_PER_TILE = NPAD // NS

CHUNK = 80
NCHUNK = 125
EPAD = NW * NCHUNK * CHUNK

_SC_MESH = plsc.VectorSubcoreMesh(
    core_axis_name="c", subcore_axis_name="s", num_cores=NC, num_subcores=NS)


@functools.partial(
    pl.kernel,
    out_type=jax.ShapeDtypeStruct((NC, NPAD), jnp.float32),
    mesh=_SC_MESH,
    scratch_types=[
        pltpu.VMEM((NCHUNK, CHUNK), jnp.int32),
        pltpu.VMEM((CHUNK,), jnp.float32),
        pltpu.VMEM_SHARED((NPAD,), jnp.float32),
        pltpu.SemaphoreType.DMA,
    ],
    compiler_params=pltpu.CompilerParams(use_tc_tiling_on_sc=False),
)
def _sc_deg(dst_hbm, zeros_hbm, ones_hbm, out_hbm, dst_v, ones_v, acc, sem):
    cid = lax.axis_index("c")
    sid = lax.axis_index("s")
    wid = cid * NS + sid
    row0 = pl.multiple_of(sid * ROWS_PER_TILE, 8)
    pltpu.sync_copy(zeros_hbm, acc.at[pl.ds(row0, ROWS_PER_TILE)])
    pltpu.sync_copy(ones_hbm, ones_v)
    pltpu.sync_copy(dst_hbm.at[wid], dst_v)
    plsc.subcore_barrier()

    def wait_one():
        pltpu.make_async_copy(ones_v, acc.at[dst_v.at[0]], sem).wait()

    def body(j, carry):
        pltpu.async_copy(ones_v, acc.at[dst_v.at[j]], sem, add=True)
        return carry

    lax.fori_loop(0, 8, body, 0)

    def body2(j, carry):
        pltpu.async_copy(ones_v, acc.at[dst_v.at[j]], sem, add=True)
        wait_one()
        return carry

    lax.fori_loop(8, NCHUNK, body2, 0)

    def drain(j, carry):
        wait_one()
        return carry

    lax.fori_loop(0, 8, drain, 0)
    plsc.subcore_barrier()
    pltpu.sync_copy(acc.at[pl.ds(row0, ROWS_PER_TILE)],
                    out_hbm.at[cid, pl.ds(row0, ROWS_PER_TILE)])


@functools.partial(
    pl.kernel,
    out_type=jax.ShapeDtypeStruct((NC, NPAD, H), jnp.float32),
    mesh=_SC_MESH,
    scratch_types=[
        pltpu.VMEM((NCHUNK, CHUNK), jnp.int32),
        pltpu.VMEM((NCHUNK, CHUNK), jnp.int32),
        [pltpu.VMEM((CHUNK, H), jnp.float32)] * 8,
        [pltpu.SemaphoreType.DMA] * 8,
        [pltpu.SemaphoreType.DMA] * 8,
        pltpu.VMEM_SHARED((NPAD, H), jnp.float32),
    ],
    compiler_params=pltpu.CompilerParams(use_tc_tiling_on_sc=False),
)
def _sc_agg(t_hbm, src_hbm, dst_hbm, zeros_hbm, out_hbm,
            src_v, dst_v, bufs, gs, ss, acc):
    cid = lax.axis_index("c")
    sid = lax.axis_index("s")
    wid = cid * NS + sid
    row0 = pl.multiple_of(sid * ROWS_PER_TILE, 8)
    pltpu.sync_copy(zeros_hbm, acc.at[pl.ds(row0, ROWS_PER_TILE)])
    pltpu.sync_copy(src_hbm.at[wid], src_v)
    pltpu.sync_copy(dst_hbm.at[wid], dst_v)
    plsc.subcore_barrier()

    def gather(j, k):
        pltpu.async_copy(t_hbm.at[src_v.at[j]], bufs[k], gs[k])

    def scatter(j, k):
        pltpu.async_copy(bufs[k], acc.at[dst_v.at[j]], ss[k], add=True)

    def wait(k, sem):
        pltpu.make_async_copy(t_hbm.at[src_v.at[0]], bufs[k], sem).wait()

    for k in range(4):
        gather(k, k)
    for p in range(8):
        k = p % 8
        k4 = (p + 4) % 8
        wait(k, gs[k])
        scatter(p, k)
        if p >= 4:
            wait(k4, ss[k4])
        gather(p + 4, k4)

    def round_body(i, carry):
        p0 = 8 * i
        for k in range(8):
            k4 = (k + 4) % 8
            wait(k, gs[k])
            scatter(p0 + k, k)
            wait(k4, ss[k4])
            gather(p0 + k + 4, k4)
        return carry

    i_end = (NCHUNK - 12) // 8 + 1
    lax.fori_loop(1, i_end, round_body, 0)

    for p in range(8 * i_end, NCHUNK):
        k = p % 8
        k4 = (p + 4) % 8
        wait(k, gs[k])
        scatter(p, k)
        wait(k4, ss[k4])
        if p + 4 < NCHUNK:
            gather(p + 4, k4)
    for c in range(NCHUNK - 4, NCHUNK):
        wait(c % 8, ss[c % 8])
    plsc.subcore_barrier()
    pltpu.sync_copy(acc.at[pl.ds(row0, ROWS_PER_TILE)],
                    out_hbm.at[cid, pl.ds(row0, ROWS_PER_TILE)])


def _tc1_body(deg0, deg1, x, W1, dis_o, t1p_o):
    deg = deg0[...] + deg1[...] + 1.0
    dis = lax.rsqrt(deg)
    dis_o[...] = dis
    t = lax.dot_general(x[...].astype(jnp.bfloat16), W1[...].astype(jnp.bfloat16),
                        (((1,), (1,)), ((), ())),
                        preferred_element_type=jnp.float32)
    t1p_o[...] = t * dis


def _tc2_body(agg0, agg1, t1p, dis, b1, gamma1, beta1, W2, t2p_o):
    d = dis[...]
    z = (agg0[...] + agg1[...] + t1p[...]) * d + b1[...]
    mu = jnp.mean(z, axis=0, keepdims=True)
    var = jnp.mean((z - mu) ** 2, axis=0, keepdims=True)
    h = jnp.maximum((z - mu) / jnp.sqrt(var + 1e-5) * gamma1[...] + beta1[...], 0.0)
    t2 = lax.dot_general(h.astype(jnp.bfloat16), W2[...].astype(jnp.bfloat16),
                         (((1,), (1,)), ((), ())),
                         preferred_element_type=jnp.float32)
    t2p_o[...] = t2 * d


def _tc3_body(agg0, agg1, t2p, dis, b2, gamma2, beta2, batch, Wl, bl, out_o):
    z = (agg0[...] + agg1[...] + t2p[...]) * dis[...] + b2[...]
    mu = jnp.mean(z, axis=0, keepdims=True)
    var = jnp.mean((z - mu) ** 2, axis=0, keepdims=True)
    h = jnp.maximum((z - mu) / jnp.sqrt(var + 1e-5) * gamma2[...] + beta2[...], 0.0)
    gid = lax.broadcasted_iota(jnp.int32, (G, N), 0)
    onehot = (batch[...] == gid).astype(jnp.float32)
    sums = lax.dot_general(onehot, h, (((1,), (0,)), ((), ())),
                           preferred_element_type=jnp.float32,
                        precision=lax.Precision.HIGHEST)
    cnt = jnp.sum(onehot, axis=1, keepdims=True)
    mean = sums / jnp.maximum(cnt, 1.0)
    mb = mean.astype(jnp.bfloat16).astype(jnp.float32)
    wb = Wl[...].astype(jnp.bfloat16).astype(jnp.float32)
    out_o[...] = jnp.sum(mb * wb, axis=1, keepdims=True) + bl[...]


def kernel(x, edge_index, batch, W1, b1, gamma1, beta1, W2, b2, gamma2, beta2,
           Wl, bl):
    src = edge_index[0].reshape(NW, NCHUNK, CHUNK)
    dst = edge_index[1].reshape(NW, NCHUNK, CHUNK)
    zeros_row = jnp.zeros((ROWS_PER_TILE,), jnp.float32)
    zeros_mat = jnp.zeros((ROWS_PER_TILE, H), jnp.float32)
    ones_chunk = jnp.ones((CHUNK,), jnp.float32)

    degp = _sc_deg(dst, zeros_row, ones_chunk)
    deg0 = degp[0, :N].reshape(N, 1)
    deg1 = degp[1, :N].reshape(N, 1)

    dis, t1p = pl.pallas_call(
        _tc1_body,
        out_shape=[
            jax.ShapeDtypeStruct((N, 1), jnp.float32),
            jax.ShapeDtypeStruct((N, H), jnp.float32),
        ],
    )(deg0, deg1, x, W1)

    agg1 = _sc_agg(t1p, src, dst, zeros_mat)

    t2p = pl.pallas_call(
        _tc2_body,
        out_shape=jax.ShapeDtypeStruct((N, H), jnp.float32),
    )(agg1[0, :N], agg1[1, :N], t1p, dis, b1, gamma1, beta1, W2)

    agg2 = _sc_agg(t2p, src, dst, zeros_mat)

    out = pl.pallas_call(
        _tc3_body,
        out_shape=jax.ShapeDtypeStruct((G, 1), jnp.float32),
    )(agg2[0, :N], agg2[1, :N], t2p, dis, b2, gamma2, beta2,
      batch.reshape(1, N), Wl, bl)
    return out


# speedup vs baseline: 2.6023x; 1.1112x over previous
import functools

import jax
import jax.numpy as jnp
from jax import lax
from jax.experimental import pallas as pl
from jax.experimental.pallas import tpu as pltpu
from jax.experimental.pallas import tpu_sc as plsc

N = 10000
E = 320000
D = 128
H = 64
G = 128

NC = 2
NS = 16
NW = NC * NS

NPAD = 10240
ROWS_PER_TILE = NPAD // NS

CHUNK = 80
NCHUNK = 125
EPAD = NW * NCHUNK * CHUNK

_SC_MESH = plsc.VectorSubcoreMesh(
    core_axis_name="c", subcore_axis_name="s", num_cores=NC, num_subcores=NS)


@functools.partial(
    pl.kernel,
    out_type=jax.ShapeDtypeStruct((NC, NPAD), jnp.float32),
    mesh=_SC_MESH,
    scratch_types=[
        pltpu.VMEM((NCHUNK, CHUNK), jnp.int32),
        pltpu.VMEM((CHUNK,), jnp.float32),
        pltpu.VMEM_SHARED((NPAD,), jnp.float32),
        pltpu.SemaphoreType.DMA,
    ],
    compiler_params=pltpu.CompilerParams(use_tc_tiling_on_sc=False),
)
def _sc_deg(ei_hbm, zeros_hbm, ones_hbm, out_hbm, dst_v, ones_v, acc, sem):
    cid = lax.axis_index("c")
    sid = lax.axis_index("s")
    wid = cid * NS + sid
    row0 = pl.multiple_of(sid * ROWS_PER_TILE, 8)
    pltpu.sync_copy(zeros_hbm, acc.at[pl.ds(row0, ROWS_PER_TILE)])
    pltpu.sync_copy(ones_hbm, ones_v)
    pltpu.sync_copy(ei_hbm.at[1, wid], dst_v)
    plsc.subcore_barrier()

    def wait_one():
        pltpu.make_async_copy(ones_v, acc.at[dst_v.at[0]], sem).wait()

    def body(j, carry):
        pltpu.async_copy(ones_v, acc.at[dst_v.at[j]], sem, add=True)
        return carry

    lax.fori_loop(0, 8, body, 0)

    def body2(j, carry):
        pltpu.async_copy(ones_v, acc.at[dst_v.at[j]], sem, add=True)
        wait_one()
        return carry

    lax.fori_loop(8, NCHUNK, body2, 0)

    def drain(j, carry):
        wait_one()
        return carry

    lax.fori_loop(0, 8, drain, 0)
    plsc.subcore_barrier()
    pltpu.sync_copy(acc.at[pl.ds(row0, ROWS_PER_TILE)],
                    out_hbm.at[cid, pl.ds(row0, ROWS_PER_TILE)])


@functools.partial(
    pl.kernel,
    out_type=jax.ShapeDtypeStruct((NC, NPAD, H), jnp.float32),
    mesh=_SC_MESH,
    scratch_types=[
        pltpu.VMEM((NCHUNK, CHUNK), jnp.int32),
        pltpu.VMEM((NCHUNK, CHUNK), jnp.int32),
        [pltpu.VMEM((CHUNK, H), jnp.float32)] * 8,
        [pltpu.SemaphoreType.DMA] * 8,
        [pltpu.SemaphoreType.DMA] * 8,
        pltpu.VMEM_SHARED((NPAD, H), jnp.float32),
    ],
    compiler_params=pltpu.CompilerParams(use_tc_tiling_on_sc=False),
)
def _sc_agg(t_hbm, ei_hbm, zeros_hbm, out_hbm,
            src_v, dst_v, bufs, gs, ss, acc):
    cid = lax.axis_index("c")
    sid = lax.axis_index("s")
    wid = cid * NS + sid
    row0 = pl.multiple_of(sid * ROWS_PER_TILE, 8)
    pltpu.sync_copy(zeros_hbm, acc.at[pl.ds(row0, ROWS_PER_TILE)])
    pltpu.sync_copy(ei_hbm.at[0, wid], src_v)
    pltpu.sync_copy(ei_hbm.at[1, wid], dst_v)
    plsc.subcore_barrier()

    def gather(j, k):
        pltpu.async_copy(t_hbm.at[src_v.at[j]], bufs[k], gs[k])

    def scatter(j, k):
        pltpu.async_copy(bufs[k], acc.at[dst_v.at[j]], ss[k], add=True)

    def wait(k, sem):
        pltpu.make_async_copy(t_hbm.at[src_v.at[0]], bufs[k], sem).wait()

    for k in range(4):
        gather(k, k)
    for p in range(8):
        k = p % 8
        k4 = (p + 4) % 8
        wait(k, gs[k])
        scatter(p, k)
        if p >= 4:
            wait(k4, ss[k4])
        gather(p + 4, k4)

    def round_body(i, carry):
        p0 = 8 * i
        for k in range(8):
            k4 = (k + 4) % 8
            wait(k, gs[k])
            scatter(p0 + k, k)
            wait(k4, ss[k4])
            gather(p0 + k + 4, k4)
        return carry

    i_end = (NCHUNK - 12) // 8 + 1
    lax.fori_loop(1, i_end, round_body, 0)

    for p in range(8 * i_end, NCHUNK):
        k = p % 8
        k4 = (p + 4) % 8
        wait(k, gs[k])
        scatter(p, k)
        wait(k4, ss[k4])
        if p + 4 < NCHUNK:
            gather(p + 4, k4)
    for c in range(NCHUNK - 4, NCHUNK):
        wait(c % 8, ss[c % 8])
    plsc.subcore_barrier()
    pltpu.sync_copy(acc.at[pl.ds(row0, ROWS_PER_TILE)],
                    out_hbm.at[cid, pl.ds(row0, ROWS_PER_TILE)])


def _tc1_body(deg0, deg1, x, W1, dis_o, t1p_o):
    deg = deg0[...] + deg1[...] + 1.0
    dis = lax.rsqrt(deg)
    dis_o[...] = dis
    t = lax.dot_general(x[...].astype(jnp.bfloat16), W1[...].astype(jnp.bfloat16),
                        (((1,), (1,)), ((), ())),
                        preferred_element_type=jnp.float32)
    t1p_o[...] = t * dis


def _tc2_body(aggp, t1p, dis, b1, gamma1, beta1, W2, t2p_o):
    d = dis[...]
    z = (aggp[0, :N, :] + aggp[1, :N, :] + t1p[...]) * d + b1[...]
    mu = jnp.mean(z, axis=0, keepdims=True)
    var = jnp.mean((z - mu) ** 2, axis=0, keepdims=True)
    h = jnp.maximum((z - mu) / jnp.sqrt(var + 1e-5) * gamma1[...] + beta1[...], 0.0)
    t2 = lax.dot_general(h.astype(jnp.bfloat16), W2[...].astype(jnp.bfloat16),
                         (((1,), (1,)), ((), ())),
                         preferred_element_type=jnp.float32)
    t2p_o[...] = t2 * d


def _tc3_body(aggp, t2p, dis, b2, gamma2, beta2, batch, Wl, bl, out_o):
    z = (aggp[0, :N, :] + aggp[1, :N, :] + t2p[...]) * dis[...] + b2[...]
    mu = jnp.mean(z, axis=0, keepdims=True)
    var = jnp.mean((z - mu) ** 2, axis=0, keepdims=True)
    h = jnp.maximum((z - mu) / jnp.sqrt(var + 1e-5) * gamma2[...] + beta2[...], 0.0)
    gid = lax.broadcasted_iota(jnp.int32, (G, N), 0)
    onehot = (batch[...] == gid).astype(jnp.float32)
    sums = lax.dot_general(onehot, h, (((1,), (0,)), ((), ())),
                           preferred_element_type=jnp.float32,
                        precision=lax.Precision.HIGHEST)
    cnt = jnp.sum(onehot, axis=1, keepdims=True)
    mean = sums / jnp.maximum(cnt, 1.0)
    mb = mean.astype(jnp.bfloat16).astype(jnp.float32)
    wb = Wl[...].astype(jnp.bfloat16).astype(jnp.float32)
    out_o[...] = jnp.sum(mb * wb, axis=1, keepdims=True) + bl[...]


def kernel(x, edge_index, batch, W1, b1, gamma1, beta1, W2, b2, gamma2, beta2,
           Wl, bl):
    ei = edge_index.reshape(2, NW, NCHUNK, CHUNK)
    zeros_row = jnp.zeros((ROWS_PER_TILE,), jnp.float32)
    zeros_mat = jnp.zeros((ROWS_PER_TILE, H), jnp.float32)
    ones_chunk = jnp.ones((CHUNK,), jnp.float32)

    degp = _sc_deg(ei, zeros_row, ones_chunk)
    deg0 = degp[0, :N].reshape(N, 1)
    deg1 = degp[1, :N].reshape(N, 1)

    dis, t1p = pl.pallas_call(
        _tc1_body,
        out_shape=[
            jax.ShapeDtypeStruct((N, 1), jnp.float32),
            jax.ShapeDtypeStruct((N, H), jnp.float32),
        ],
    )(deg0, deg1, x, W1)

    agg1 = _sc_agg(t1p, ei, zeros_mat)

    t2p = pl.pallas_call(
        _tc2_body,
        out_shape=jax.ShapeDtypeStruct((N, H), jnp.float32),
    )(agg1, t1p, dis, b1, gamma1, beta1, W2)

    agg2 = _sc_agg(t2p, ei, zeros_mat)

    out = pl.pallas_call(
        _tc3_body,
        out_shape=jax.ShapeDtypeStruct((G, 1), jnp.float32),
    )(agg2, t2p, dis, b2, gamma2, beta2,
      batch.reshape(1, N), Wl, bl)
    return out
